# baseline (speedup 1.0000x reference)
import functools

import jax
import jax.numpy as jnp
from jax import lax
from jax.experimental import pallas as pl
from jax.experimental.pallas import tpu as pltpu
from jax.experimental.pallas import tpu_sc as plsc

NC = 2
NS = 16
LANES = 16


def _sc_propagate(x, src, dst, w):
  N, F = x.shape
  E = src.shape[0]
  NPAD = ((N + NS * LANES - 1) // (NS * LANES)) * NS * LANES
  NPT = NPAD // NS
  XPT = ((N + NS * 8 - 1) // (NS * 8)) * 8
  NXP = XPT * NS
  EC = E // NC
  ET = EC // NS
  CH = 80
  NFULL = ET // CH
  REM = ET - NFULL * CH
  DEG_PT = E // NS
  DCH = 2000
  assert DCH % LANES == 0 and DEG_PT % DCH == 0
  NDCH = DEG_PT // DCH

  mesh = plsc.VectorSubcoreMesh(core_axis_name="c", subcore_axis_name="s")

  NPR = NPAD // F
  RPT = NPR // NS

  scratch_types = [
      pltpu.VMEM_SHARED((NPR, F), jnp.float32),
      pltpu.VMEM_SHARED((NPAD,), jnp.float32),
      pltpu.VMEM_SHARED((NXP, F), jnp.float32),
      pltpu.VMEM((RPT, F), jnp.float32),
      pltpu.VMEM((NPT,), jnp.float32),
      pltpu.VMEM((NPAD,), jnp.float32),
      pltpu.VMEM((NPR,), jnp.int32),
      pltpu.VMEM((CH,), jnp.int32),
      pltpu.VMEM((CH,), jnp.int32),
      pltpu.VMEM((CH,), jnp.float32),
      pltpu.VMEM((CH,), jnp.float32),
      pltpu.VMEM((CH, F), jnp.float32),
      pltpu.SemaphoreType.DMA,
      pltpu.SemaphoreType.DMA,
      pltpu.VMEM((CH,), jnp.int32),
      pltpu.VMEM((CH,), jnp.int32),
      pltpu.VMEM((CH,), jnp.float32),
      pltpu.VMEM((CH,), jnp.float32),
      pltpu.VMEM((CH, F), jnp.float32),
      pltpu.SemaphoreType.DMA,
      pltpu.SemaphoreType.DMA,
  ]

  @functools.partial(
      pl.kernel,
      out_type=(
          jax.ShapeDtypeStruct((NC, NXP, F), jnp.float32),
          jax.ShapeDtypeStruct((NPAD,), jnp.float32),
      ),
      mesh=mesh,
      compiler_params=pltpu.CompilerParams(needs_layout_passes=False),
      scratch_types=scratch_types,
  )
  def body(x_h, src_h, dst_h, w_h, xpp_h, dinv_h,
           sdeg, sdinv, sxp, dbuf, dacc, dlocal, riota,
           sidx0, didx0, wbuf0, nbuf0, rows0, gsem0, ssem0,
           sidx1, didx1, wbuf1, nbuf1, rows1, gsem1, ssem1):
    c = lax.axis_index("c")
    s = lax.axis_index("s")
    zero16 = jnp.zeros((LANES,), jnp.float32)
    bufs = ((sidx0, didx0, wbuf0, nbuf0, rows0, gsem0, ssem0),
            (sidx1, didx1, wbuf1, nbuf1, rows1, gsem1, ssem1))

    for rows in (rows0, rows1):
      @pl.loop(0, CH)
      def _(r, rows=rows):
        for j in range(F // LANES):
          rows[r, pl.ds(j * LANES, LANES)] = zero16

    @pl.loop(0, NPR // LANES)
    def _(i):
      riota[pl.ds(i * LANES, LANES)] = \
          lax.iota(jnp.int32, LANES) + i * LANES

    pltpu.sync_copy(rows0.at[pl.ds(0, RPT)], sdeg.at[pl.ds(s * RPT, RPT)])

    def phase_a(ldeg, didxL, dwL):
      @pl.loop(0, NPR)
      def _(r):
        for j in range(F // LANES):
          ldeg[r, pl.ds(j * LANES, LANES)] = zero16

      @pl.loop(0, NDCH)
      def _(k):
        base = s * DEG_PT + k * DCH
        pltpu.sync_copy(dst_h.at[pl.ds(base, DCH)], didxL)
        pltpu.sync_copy(w_h.at[pl.ds(base, DCH)], dwL)

        @pl.loop(0, DCH // LANES)
        def _(i):
          dv = didxL[pl.ds(i * LANES, LANES)]
          wv = dwL[pl.ds(i * LANES, LANES)]
          plsc.addupdate_scatter(ldeg, [dv >> 7, dv & 127], wv)

      plsc.subcore_barrier()
      pltpu.sync_copy(ldeg, sdeg.at[riota], add=True)

    pl.run_scoped(phase_a,
                  pltpu.VMEM((NPR, F), jnp.float32),
                  pltpu.VMEM((DCH,), jnp.int32),
                  pltpu.VMEM((DCH,), jnp.float32))
    plsc.subcore_barrier()

    pltpu.sync_copy(sdeg.at[pl.ds(s * RPT, RPT)], dbuf)

    @pl.loop(0, RPT)
    def _(r):
      for j in range(F // LANES):
        v = dbuf[r, pl.ds(j * LANES, LANES)] + 1.0
        bi = plsc.bitcast(v, jnp.int32)
        y = plsc.bitcast(jnp.int32(0x5F3759DF) - (bi >> 1), jnp.float32)
        y = y * (1.5 - 0.5 * v * y * y)
        y = y * (1.5 - 0.5 * v * y * y)
        y = y * (1.5 - 0.5 * v * y * y)
        dacc[pl.ds(r * F + j * LANES, LANES)] = y

    pltpu.sync_copy(dacc, sdinv.at[pl.ds(s * NPT, NPT)])

    @pl.when(c == 0)
    def _():
      pltpu.sync_copy(dacc, dinv_h.at[pl.ds(s * NPT, NPT)])

    plsc.subcore_barrier()
    pltpu.sync_copy(sdinv, dlocal)

    r0 = s * XPT
    for kk in range(XPT // CH):
      pltpu.sync_copy(rows0, sxp.at[pl.ds(r0 + kk * CH, CH)])
    if XPT % CH:
      pltpu.sync_copy(rows0.at[pl.ds(0, XPT % CH)],
                      sxp.at[pl.ds(r0 + (XPT // CH) * CH, XPT % CH)])
    plsc.subcore_barrier()

    eb = c * EC + s * ET

    def load_idx(k, sidx, didx, wbuf):
      eo = eb + k * CH
      pltpu.sync_copy(src_h.at[pl.ds(eo, CH)], sidx)
      pltpu.sync_copy(dst_h.at[pl.ds(eo, CH)], didx)
      pltpu.sync_copy(w_h.at[pl.ds(eo, CH)], wbuf)

    def fire_gather(sidx, rows, gsem):
      pltpu.async_copy(x_h.at[sidx], rows, gsem)

    def wait_gather(sidx, rows, gsem):
      pltpu.make_async_copy(x_h.at[sidx], rows, gsem).wait()

    def fire_scatter(rows, didx, ssem):
      pltpu.async_copy(rows, sxp.at[didx], ssem, add=True)

    def wait_scatter(rows, didx, ssem):
      pltpu.make_async_copy(rows, sxp.at[didx], ssem).wait()

    def compute(sidx, didx, wbuf, nbuf, rows):
      @pl.loop(0, CH // LANES)
      def _(i):
        sl = pl.ds(i * LANES, LANES)
        nbuf[sl] = plsc.load_gather(dlocal, [sidx[sl]]) * wbuf[sl] * \
            plsc.load_gather(dlocal, [didx[sl]])

      @pl.loop(0, CH // LANES)
      def _(g):
        nv = nbuf[pl.ds(g * LANES, LANES)]
        for l in range(LANES):
          nval = nv[l]
          r = g * LANES + l
          for j in range(F // LANES):
            sl = pl.ds(j * LANES, LANES)
            rows[r, sl] = rows[r, sl] * nval

    load_idx(0, sidx0, didx0, wbuf0)
    fire_gather(sidx0, rows0, gsem0)

    @pl.loop(0, NFULL // 2)
    def _(p):
      for parity in (0, 1):
        k = p * 2 + parity
        sidx, didx, wbuf, nbuf, rows, gsem, ssem = bufs[parity]
        sidx2, didx2, wbuf2, nbuf2, rows2, gsem2, ssem2 = bufs[1 - parity]
        wait_gather(sidx, rows, gsem)
        compute(sidx, didx, wbuf, nbuf, rows)
        fire_scatter(rows, didx, ssem)

        @pl.when(k > 0)
        def _():
          wait_scatter(rows2, didx2, ssem2)

        @pl.when(k + 1 < NFULL)
        def _():
          load_idx(k + 1, sidx2, didx2, wbuf2)
          fire_gather(sidx2, rows2, gsem2)

    if NFULL % 2 == 1:
      sidx, didx, wbuf, nbuf, rows, gsem, ssem = bufs[(NFULL - 1) % 2]
      sidx2, didx2, wbuf2, nbuf2, rows2, gsem2, ssem2 = bufs[NFULL % 2]
      wait_gather(sidx, rows, gsem)
      compute(sidx, didx, wbuf, nbuf, rows)
      pltpu.sync_copy(rows, sxp.at[didx], add=True)
      wait_scatter(rows2, didx2, ssem2)
    else:
      wait_scatter(rows0, didx0, ssem0)
      wait_scatter(rows1, didx1, ssem1)

    if REM:
      eo = eb + NFULL * CH
      pltpu.sync_copy(src_h.at[pl.ds(eo, REM)], sidx0.at[pl.ds(0, REM)])
      pltpu.sync_copy(dst_h.at[pl.ds(eo, REM)], didx0.at[pl.ds(0, REM)])
      pltpu.sync_copy(w_h.at[pl.ds(eo, REM)], wbuf0.at[pl.ds(0, REM)])
      for i in range(REM // LANES, CH // LANES):
        wbuf0[pl.ds(i * LANES, LANES)] = zero16
      fire_gather(sidx0, rows0, gsem0)
      wait_gather(sidx0, rows0, gsem0)
      compute(sidx0, didx0, wbuf0, nbuf0, rows0)
      pltpu.sync_copy(rows0, sxp.at[didx0], add=True)

    plsc.subcore_barrier()

    rr = s * XPT
    pltpu.sync_copy(sxp.at[pl.ds(rr, XPT)], xpp_h.at[c, pl.ds(rr, XPT)])

  return body(x, src, dst, w)


def _tc_head(xpp, x, dinv_n, Wz, bz, Wlz, blz, Wh, bh, Wlh, blh, Wout, bout):
  N, F = x.shape
  B = 1000
  NB = N // B

  def body(xpp_ref, x_ref, dinv_ref, wz_ref, bz_ref, wlz_ref, blz_ref,
           wh_ref, bh_ref, wlh_ref, blh_ref, wout_ref, bout_ref, y_ref,
           mz_s, cz_s, mh_s, ch_s):
    @pl.when(pl.program_id(0) == 0)
    def _():
      wlz_t = wlz_ref[0:F, :]
      wlh_t = wlh_ref[0:F, :]
      mz_s[...] = jnp.dot(wz_ref[...], wlz_t, preferred_element_type=jnp.float32)
      cz_s[...] = jnp.dot(bz_ref[...], wlz_t, preferred_element_type=jnp.float32) + blz_ref[...]
      mh_s[...] = jnp.dot(wh_ref[...], wlh_t, preferred_element_type=jnp.float32)
      ch_s[...] = jnp.dot(bh_ref[...], wlh_t, preferred_element_type=jnp.float32) + blh_ref[...]

    d = dinv_ref[...]
    xp = xpp_ref[0] + xpp_ref[1] + d * d * x_ref[...]
    z = jax.nn.sigmoid(jnp.dot(xp, mz_s[...], preferred_element_type=jnp.float32) + cz_s[...])
    ht = jnp.tanh(jnp.dot(xp, mh_s[...], preferred_element_type=jnp.float32) + ch_s[...])
    h = (1.0 - z) * ht
    y_ref[...] = jnp.dot(jnp.maximum(h, 0.0), wout_ref[...],
                         preferred_element_type=jnp.float32) + bout_ref[...]

  full = lambda shape: pl.BlockSpec(shape, lambda i: (0,) * len(shape))
  return pl.pallas_call(
      body,
      grid=(NB,),
      in_specs=[
          pl.BlockSpec((NC, B, F), lambda i: (0, i, 0)),
          pl.BlockSpec((B, F), lambda i: (i, 0)),
          pl.BlockSpec((B, 1), lambda i: (i, 0)),
          full((F, F)), full((1, F)), full((2 * F, F)), full((1, F)),
          full((F, F)), full((1, F)), full((2 * F, F)), full((1, F)),
          full((F, 1)), full((1, 1)),
      ],
      out_specs=pl.BlockSpec((B, 1), lambda i: (i, 0)),
      out_shape=jax.ShapeDtypeStruct((N, 1), jnp.float32),
      scratch_shapes=[
          pltpu.VMEM((F, F), jnp.float32),
          pltpu.VMEM((1, F), jnp.float32),
          pltpu.VMEM((F, F), jnp.float32),
          pltpu.VMEM((1, F), jnp.float32),
      ],
  )(xpp, x, dinv_n, Wz, bz, Wlz, blz, Wh, bh, Wlh, blh, Wout, bout)


def kernel(x, edge_index, edge_weight, Wz, bz, Wlz, blz, Wr, br, Wlr, blr,
           Wh, bh, Wlh, blh, Wout, bout):
  N, F = x.shape
  src = edge_index[0]
  dst = edge_index[1]
  xpp, dinv_pad = _sc_propagate(x, src, dst, edge_weight)
  xpp = xpp[:, :N]
  dinv_n = dinv_pad[:N].reshape(N, 1)
  return _tc_head(xpp, x, dinv_n,
                  Wz, bz.reshape(1, F), Wlz, blz.reshape(1, F),
                  Wh, bh.reshape(1, F), Wlh, blh.reshape(1, F),
                  Wout, bout.reshape(1, 1))


# speedup vs baseline: 1.3940x; 1.3940x over previous
import functools

import jax
import jax.numpy as jnp
from jax import lax
from jax.experimental import pallas as pl
from jax.experimental.pallas import tpu as pltpu
from jax.experimental.pallas import tpu_sc as plsc

NC = 2
NS = 16
LANES = 16


def _sc_propagate(x, src, dst, w):
  N, F = x.shape
  E = src.shape[0]
  NPAD = ((N + NS * LANES - 1) // (NS * LANES)) * NS * LANES
  NPT = NPAD // NS
  XPT = ((N + NS * 8 - 1) // (NS * 8)) * 8
  NXP = XPT * NS
  EC = E // NC
  ET = EC // NS
  CH = 128
  NFULL = ET // CH
  REM = ET - NFULL * CH
  assert NFULL % 2 == 0 and REM % 8 == 0
  DEG_PT = E // NS
  DCH = 2000
  assert DCH % LANES == 0 and DEG_PT % DCH == 0
  NDCH = DEG_PT // DCH
  NPR = NPAD // F
  RPT = NPR // NS

  mesh = plsc.VectorSubcoreMesh(core_axis_name="c", subcore_axis_name="s")

  @functools.partial(
      pl.kernel,
      out_type=(
          jax.ShapeDtypeStruct((NC, NXP, F), jnp.float32),
          jax.ShapeDtypeStruct((NPAD,), jnp.float32),
      ),
      mesh=mesh,
      compiler_params=pltpu.CompilerParams(needs_layout_passes=False),
      scratch_types=[
          pltpu.VMEM_SHARED((NPR, F), jnp.float32),
          pltpu.VMEM_SHARED((NPAD,), jnp.float32),
          pltpu.VMEM_SHARED((NXP, F), jnp.float32),
          pltpu.VMEM((RPT, F), jnp.float32),
          pltpu.VMEM((NPT,), jnp.float32),
          pltpu.VMEM((NPAD,), jnp.float32),
          pltpu.VMEM((NPR,), jnp.int32),
          pltpu.SemaphoreType.DMA,
          pltpu.SemaphoreType.DMA,
          pltpu.SemaphoreType.DMA,
          pltpu.SemaphoreType.DMA,
      ],
  )
  def body(x_h, src_h, dst_h, w_h, xpp_h, dinv_h,
           sdeg, sdinv, sxp, dbuf, dacc, dlocal, riota,
           gsem0, ssem0, gsem1, ssem1):
    c = lax.axis_index("c")
    s = lax.axis_index("s")
    zero16 = jnp.zeros((LANES,), jnp.float32)

    @pl.loop(0, NPR // LANES)
    def _(i):
      riota[pl.ds(i * LANES, LANES)] = \
          lax.iota(jnp.int32, LANES) + i * LANES

    def phase_a(ldeg, didxL, dwL):
      @pl.loop(0, NPR)
      def _(r):
        for j in range(F // LANES):
          ldeg[r, pl.ds(j * LANES, LANES)] = zero16

      pltpu.sync_copy(ldeg.at[pl.ds(0, RPT)], sdeg.at[pl.ds(s * RPT, RPT)])

      @pl.loop(0, NDCH)
      def _(k):
        base = s * DEG_PT + k * DCH
        pltpu.sync_copy(dst_h.at[pl.ds(base, DCH)], didxL)
        pltpu.sync_copy(w_h.at[pl.ds(base, DCH)], dwL)

        @pl.loop(0, DCH // LANES)
        def _(i):
          dv = didxL[pl.ds(i * LANES, LANES)]
          wv = dwL[pl.ds(i * LANES, LANES)]
          plsc.addupdate_scatter(ldeg, [dv >> 7, dv & 127], wv)

      plsc.subcore_barrier()
      pltpu.sync_copy(ldeg, sdeg.at[riota], add=True)

    pl.run_scoped(phase_a,
                  pltpu.VMEM((NPR, F), jnp.float32),
                  pltpu.VMEM((DCH,), jnp.int32),
                  pltpu.VMEM((DCH,), jnp.float32))
    plsc.subcore_barrier()

    pltpu.sync_copy(sdeg.at[pl.ds(s * RPT, RPT)], dbuf)

    @pl.loop(0, RPT)
    def _(r):
      for j in range(F // LANES):
        v = dbuf[r, pl.ds(j * LANES, LANES)] + 1.0
        bi = plsc.bitcast(v, jnp.int32)
        y = plsc.bitcast(jnp.int32(0x5F3759DF) - (bi >> 1), jnp.float32)
        y = y * (1.5 - 0.5 * v * y * y)
        y = y * (1.5 - 0.5 * v * y * y)
        y = y * (1.5 - 0.5 * v * y * y)
        dacc[pl.ds(r * F + j * LANES, LANES)] = y

    pltpu.sync_copy(dacc, sdinv.at[pl.ds(s * NPT, NPT)])

    @pl.when(c == 0)
    def _():
      pltpu.sync_copy(dacc, dinv_h.at[pl.ds(s * NPT, NPT)])

    plsc.subcore_barrier()
    pltpu.sync_copy(sdinv, dlocal)

    eb = c * EC + s * ET

    def phase_c(sidx0, didx0, wbuf0, nbuf0, rows0,
                sidx1, didx1, wbuf1, nbuf1, rows1):
      bufs = ((sidx0, didx0, wbuf0, nbuf0, rows0, gsem0, ssem0),
              (sidx1, didx1, wbuf1, nbuf1, rows1, gsem1, ssem1))

      @pl.loop(0, CH)
      def _(r):
        for j in range(F // LANES):
          rows0[r, pl.ds(j * LANES, LANES)] = zero16

      r0 = s * XPT
      for kk in range(XPT // CH):
        pltpu.sync_copy(rows0, sxp.at[pl.ds(r0 + kk * CH, CH)])
      if XPT % CH:
        pltpu.sync_copy(rows0.at[pl.ds(0, XPT % CH)],
                        sxp.at[pl.ds(r0 + (XPT // CH) * CH, XPT % CH)])
      plsc.subcore_barrier()

      def load_idx(k, sidx, didx, wbuf):
        eo = eb + k * CH
        pltpu.sync_copy(src_h.at[pl.ds(eo, CH)], sidx)
        pltpu.sync_copy(dst_h.at[pl.ds(eo, CH)], didx)
        pltpu.sync_copy(w_h.at[pl.ds(eo, CH)], wbuf)

      def fire_gather(sidx, rows, gsem):
        pltpu.async_copy(x_h.at[sidx], rows, gsem)

      def wait_gather(sidx, rows, gsem):
        pltpu.make_async_copy(x_h.at[sidx], rows, gsem).wait()

      def fire_scatter(rows, didx, ssem):
        pltpu.async_copy(rows, sxp.at[didx], ssem, add=True)

      def wait_scatter(rows, didx, ssem):
        pltpu.make_async_copy(rows, sxp.at[didx], ssem).wait()

      def compute(sidx, didx, wbuf, nbuf, rows):
        @pl.loop(0, CH // LANES)
        def _(i):
          sl = pl.ds(i * LANES, LANES)
          nbuf[sl] = plsc.load_gather(dlocal, [sidx[sl]]) * wbuf[sl] * \
              plsc.load_gather(dlocal, [didx[sl]])

        @pl.loop(0, CH // LANES)
        def _(g):
          nv = nbuf[pl.ds(g * LANES, LANES)]
          for l in range(LANES):
            nval = nv[l]
            r = g * LANES + l
            for j in range(F // LANES):
              sl = pl.ds(j * LANES, LANES)
              rows[r, sl] = rows[r, sl] * nval

      load_idx(0, sidx0, didx0, wbuf0)
      fire_gather(sidx0, rows0, gsem0)

      @pl.loop(0, NFULL // 2)
      def _(p):
        for parity in (0, 1):
          k = p * 2 + parity
          sidx, didx, wbuf, nbuf, rows, gsem, ssem = bufs[parity]
          sidx2, didx2, wbuf2, nbuf2, rows2, gsem2, ssem2 = \
              bufs[1 - parity]
          wait_gather(sidx, rows, gsem)

          @pl.when(k > 0)
          def _():
            wait_scatter(rows2, didx2, ssem2)

          @pl.when(k + 1 < NFULL)
          def _():
            load_idx(k + 1, sidx2, didx2, wbuf2)
            fire_gather(sidx2, rows2, gsem2)

          compute(sidx, didx, wbuf, nbuf, rows)
          fire_scatter(rows, didx, ssem)

      wait_scatter(rows1, didx1, ssem1)

      if REM:
        eo = eb + NFULL * CH
        pltpu.sync_copy(src_h.at[pl.ds(eo, REM)], sidx0.at[pl.ds(0, REM)])
        pltpu.sync_copy(dst_h.at[pl.ds(eo, REM)], didx0.at[pl.ds(0, REM)])
        pltpu.sync_copy(w_h.at[pl.ds(eo, REM)], wbuf0.at[pl.ds(0, REM)])
        for i in range(REM // LANES, CH // LANES):
          wbuf0[pl.ds(i * LANES, LANES)] = zero16
        fire_gather(sidx0, rows0, gsem0)
        wait_gather(sidx0, rows0, gsem0)
        compute(sidx0, didx0, wbuf0, nbuf0, rows0)
        pltpu.sync_copy(rows0, sxp.at[didx0], add=True)

    pl.run_scoped(phase_c,
                  pltpu.VMEM((CH,), jnp.int32),
                  pltpu.VMEM((CH,), jnp.int32),
                  pltpu.VMEM((CH,), jnp.float32),
                  pltpu.VMEM((CH,), jnp.float32),
                  pltpu.VMEM((CH, F), jnp.float32),
                  pltpu.VMEM((CH,), jnp.int32),
                  pltpu.VMEM((CH,), jnp.int32),
                  pltpu.VMEM((CH,), jnp.float32),
                  pltpu.VMEM((CH,), jnp.float32),
                  pltpu.VMEM((CH, F), jnp.float32))

    plsc.subcore_barrier()

    rr = s * XPT
    pltpu.sync_copy(sxp.at[pl.ds(rr, XPT)], xpp_h.at[c, pl.ds(rr, XPT)])

  return body(x, src, dst, w)


def _tc_head(xpp, x, dinv_n, Wz, bz, Wlz, blz, Wh, bh, Wlh, blh, Wout, bout):
  N, F = x.shape
  B = 1000
  NB = N // B

  def body(xpp_ref, x_ref, dinv_ref, wz_ref, bz_ref, wlz_ref, blz_ref,
           wh_ref, bh_ref, wlh_ref, blh_ref, wout_ref, bout_ref, y_ref,
           mz_s, cz_s, mh_s, ch_s):
    @pl.when(pl.program_id(0) == 0)
    def _():
      wlz_t = wlz_ref[0:F, :]
      wlh_t = wlh_ref[0:F, :]
      mz_s[...] = jnp.dot(wz_ref[...], wlz_t, preferred_element_type=jnp.float32)
      cz_s[...] = jnp.dot(bz_ref[...], wlz_t, preferred_element_type=jnp.float32) + blz_ref[...]
      mh_s[...] = jnp.dot(wh_ref[...], wlh_t, preferred_element_type=jnp.float32)
      ch_s[...] = jnp.dot(bh_ref[...], wlh_t, preferred_element_type=jnp.float32) + blh_ref[...]

    d = dinv_ref[...]
    xp = xpp_ref[0] + xpp_ref[1] + d * d * x_ref[...]
    z = jax.nn.sigmoid(jnp.dot(xp, mz_s[...], preferred_element_type=jnp.float32) + cz_s[...])
    ht = jnp.tanh(jnp.dot(xp, mh_s[...], preferred_element_type=jnp.float32) + ch_s[...])
    h = (1.0 - z) * ht
    y_ref[...] = jnp.dot(jnp.maximum(h, 0.0), wout_ref[...],
                         preferred_element_type=jnp.float32) + bout_ref[...]

  full = lambda shape: pl.BlockSpec(shape, lambda i: (0,) * len(shape))
  return pl.pallas_call(
      body,
      grid=(NB,),
      in_specs=[
          pl.BlockSpec((NC, B, F), lambda i: (0, i, 0)),
          pl.BlockSpec((B, F), lambda i: (i, 0)),
          pl.BlockSpec((B, 1), lambda i: (i, 0)),
          full((F, F)), full((1, F)), full((2 * F, F)), full((1, F)),
          full((F, F)), full((1, F)), full((2 * F, F)), full((1, F)),
          full((F, 1)), full((1, 1)),
      ],
      out_specs=pl.BlockSpec((B, 1), lambda i: (i, 0)),
      out_shape=jax.ShapeDtypeStruct((N, 1), jnp.float32),
      scratch_shapes=[
          pltpu.VMEM((F, F), jnp.float32),
          pltpu.VMEM((1, F), jnp.float32),
          pltpu.VMEM((F, F), jnp.float32),
          pltpu.VMEM((1, F), jnp.float32),
      ],
  )(xpp, x, dinv_n, Wz, bz, Wlz, blz, Wh, bh, Wlh, blh, Wout, bout)


def kernel(x, edge_index, edge_weight, Wz, bz, Wlz, blz, Wr, br, Wlr, blr,
           Wh, bh, Wlh, blh, Wout, bout):
  N, F = x.shape
  src = edge_index[0]
  dst = edge_index[1]
  xpp, dinv_pad = _sc_propagate(x, src, dst, edge_weight)
  xpp = xpp[:, :N]
  dinv_n = dinv_pad[:N].reshape(N, 1)
  return _tc_head(xpp, x, dinv_n,
                  Wz, bz.reshape(1, F), Wlz, blz.reshape(1, F),
                  Wh, bh.reshape(1, F), Wlh, blh.reshape(1, F),
                  Wout, bout.reshape(1, 1))


# speedup vs baseline: 1.9796x; 1.4201x over previous
import functools

import jax
import jax.numpy as jnp
from jax import lax
from jax.experimental import pallas as pl
from jax.experimental.pallas import tpu as pltpu
from jax.experimental.pallas import tpu_sc as plsc

NC = 2
NS = 16
LANES = 16


def _sc_propagate(x, src, dst, w):
  N, F = x.shape
  E = src.shape[0]
  NPAD = ((N + NS * LANES - 1) // (NS * LANES)) * NS * LANES
  NPT = NPAD // NS
  XPT = ((N + NS * 8 - 1) // (NS * 8)) * 8
  NXP = XPT * NS
  EC = E // NC
  ET = EC // NS
  CH = 128
  NFULL = ET // CH
  REM = ET - NFULL * CH
  assert NFULL % 6 == 0 and REM % 8 == 0
  DEG_PT = E // NS
  DCH = 2000
  assert DCH % LANES == 0 and DEG_PT % DCH == 0
  NDCH = DEG_PT // DCH
  NPR = NPAD // F
  RPT = NPR // NS

  mesh = plsc.VectorSubcoreMesh(core_axis_name="c", subcore_axis_name="s")

  @functools.partial(
      pl.kernel,
      out_type=(
          jax.ShapeDtypeStruct((NC, NXP, F), jnp.float32),
          jax.ShapeDtypeStruct((NPAD,), jnp.float32),
      ),
      mesh=mesh,
      compiler_params=pltpu.CompilerParams(needs_layout_passes=False),
      scratch_types=[
          pltpu.VMEM_SHARED((NPR, F), jnp.float32),
          pltpu.VMEM_SHARED((NPAD,), jnp.float32),
          pltpu.VMEM_SHARED((NXP, F), jnp.float32),
          pltpu.VMEM((RPT, F), jnp.float32),
          pltpu.VMEM((NPT,), jnp.float32),
          pltpu.VMEM((NPAD,), jnp.float32),
          pltpu.VMEM((NPR,), jnp.int32),
          pltpu.SemaphoreType.DMA,
          pltpu.SemaphoreType.DMA,
          pltpu.SemaphoreType.DMA,
          pltpu.SemaphoreType.DMA,
          pltpu.SemaphoreType.DMA,
          pltpu.SemaphoreType.DMA,
          pltpu.SemaphoreType.DMA,
      ],
  )
  def body(x_h, src_h, dst_h, w_h, xpp_h, dinv_h,
           sdeg, sdinv, sxp, dbuf, dacc, dlocal, riota,
           gsem0, ssem0, gsem1, ssem1, isem0, isem1, isem2):
    c = lax.axis_index("c")
    s = lax.axis_index("s")
    zero16 = jnp.zeros((LANES,), jnp.float32)

    @pl.loop(0, NPR // LANES)
    def _(i):
      riota[pl.ds(i * LANES, LANES)] = \
          lax.iota(jnp.int32, LANES) + i * LANES

    def phase_a(ldeg, didxL, dwL):
      @pl.loop(0, NPR)
      def _(r):
        for j in range(F // LANES):
          ldeg[r, pl.ds(j * LANES, LANES)] = zero16

      pltpu.sync_copy(ldeg.at[pl.ds(0, RPT)], sdeg.at[pl.ds(s * RPT, RPT)])

      @pl.loop(0, NDCH)
      def _(k):
        base = s * DEG_PT + k * DCH
        pltpu.sync_copy(dst_h.at[pl.ds(base, DCH)], didxL)
        pltpu.sync_copy(w_h.at[pl.ds(base, DCH)], dwL)

        @pl.loop(0, DCH // LANES)
        def _(i):
          dv = didxL[pl.ds(i * LANES, LANES)]
          wv = dwL[pl.ds(i * LANES, LANES)]
          plsc.addupdate_scatter(ldeg, [dv >> 7, dv & 127], wv)

      plsc.subcore_barrier()
      pltpu.sync_copy(ldeg, sdeg.at[riota], add=True)

    pl.run_scoped(phase_a,
                  pltpu.VMEM((NPR, F), jnp.float32),
                  pltpu.VMEM((DCH,), jnp.int32),
                  pltpu.VMEM((DCH,), jnp.float32))
    plsc.subcore_barrier()

    pltpu.sync_copy(sdeg.at[pl.ds(s * RPT, RPT)], dbuf)

    @pl.loop(0, RPT)
    def _(r):
      for j in range(F // LANES):
        v = dbuf[r, pl.ds(j * LANES, LANES)] + 1.0
        bi = plsc.bitcast(v, jnp.int32)
        y = plsc.bitcast(jnp.int32(0x5F3759DF) - (bi >> 1), jnp.float32)
        y = y * (1.5 - 0.5 * v * y * y)
        y = y * (1.5 - 0.5 * v * y * y)
        y = y * (1.5 - 0.5 * v * y * y)
        dacc[pl.ds(r * F + j * LANES, LANES)] = y

    pltpu.sync_copy(dacc, sdinv.at[pl.ds(s * NPT, NPT)])

    @pl.when(c == 0)
    def _():
      pltpu.sync_copy(dacc, dinv_h.at[pl.ds(s * NPT, NPT)])

    plsc.subcore_barrier()
    pltpu.sync_copy(sdinv, dlocal)

    eb = c * EC + s * ET

    def phase_c(sidx0, didx0, wbuf0, sidx1, didx1, wbuf1,
                sidx2, didx2, wbuf2, nbuf, rows0, rows1):
      idxsets = ((sidx0, didx0, wbuf0, isem0),
                 (sidx1, didx1, wbuf1, isem1),
                 (sidx2, didx2, wbuf2, isem2))
      rowsets = ((rows0, gsem0, ssem0), (rows1, gsem1, ssem1))

      @pl.loop(0, CH)
      def _(r):
        for j in range(F // LANES):
          rows0[r, pl.ds(j * LANES, LANES)] = zero16

      r0 = s * XPT
      for kk in range(XPT // CH):
        pltpu.sync_copy(rows0, sxp.at[pl.ds(r0 + kk * CH, CH)])
      if XPT % CH:
        pltpu.sync_copy(rows0.at[pl.ds(0, XPT % CH)],
                        sxp.at[pl.ds(r0 + (XPT // CH) * CH, XPT % CH)])
      plsc.subcore_barrier()

      def fire_idx(k, sidx, didx, wbuf, isem):
        eo = eb + k * CH
        pltpu.async_copy(src_h.at[pl.ds(eo, CH)], sidx, isem)
        pltpu.async_copy(dst_h.at[pl.ds(eo, CH)], didx, isem)
        pltpu.async_copy(w_h.at[pl.ds(eo, CH)], wbuf, isem)

      def wait_idx(k, sidx, didx, wbuf, isem):
        eo = eb + k * CH
        pltpu.make_async_copy(src_h.at[pl.ds(eo, CH)], sidx, isem).wait()
        pltpu.make_async_copy(dst_h.at[pl.ds(eo, CH)], didx, isem).wait()
        pltpu.make_async_copy(w_h.at[pl.ds(eo, CH)], wbuf, isem).wait()

      def fire_gather(sidx, rows, gsem):
        pltpu.async_copy(x_h.at[sidx], rows, gsem)

      def wait_gather(sidx, rows, gsem):
        pltpu.make_async_copy(x_h.at[sidx], rows, gsem).wait()

      def fire_scatter(rows, didx, ssem):
        pltpu.async_copy(rows, sxp.at[didx], ssem, add=True)

      def wait_scatter(rows, didx, ssem):
        pltpu.make_async_copy(rows, sxp.at[didx], ssem).wait()

      def compute(sidx, didx, wbuf, rows):
        @pl.loop(0, CH // LANES)
        def _(i):
          sl = pl.ds(i * LANES, LANES)
          nbuf[sl] = plsc.load_gather(dlocal, [sidx[sl]]) * wbuf[sl] * \
              plsc.load_gather(dlocal, [didx[sl]])

        @pl.loop(0, CH // LANES)
        def _(g):
          nv = nbuf[pl.ds(g * LANES, LANES)]
          for l in range(LANES):
            nval = nv[l]
            r = g * LANES + l
            for j in range(F // LANES):
              sl = pl.ds(j * LANES, LANES)
              rows[r, sl] = rows[r, sl] * nval

      fire_idx(0, *idxsets[0])
      wait_idx(0, *idxsets[0])
      fire_gather(sidx0, rows0, gsem0)
      fire_idx(1, *idxsets[1])

      @pl.loop(0, NFULL // 6)
      def _(p):
        for u in range(6):
          k = p * 6 + u
          sidx, didx, wbuf, isem = idxsets[u % 3]
          sidxN, didxN, wbufN, isemN = idxsets[(u + 1) % 3]
          sidxN2, didxN2, wbufN2, isemN2 = idxsets[(u + 2) % 3]
          rows, gsem, ssem = rowsets[u % 2]
          rows2, gsem2, ssem2 = rowsets[(u + 1) % 2]

          wait_gather(sidx, rows, gsem)

          @pl.when(k > 0)
          def _():
            wait_scatter(rows2, didxN2, ssem2)

          @pl.when(k + 1 < NFULL)
          def _():
            wait_idx(k + 1, sidxN, didxN, wbufN, isemN)
            fire_gather(sidxN, rows2, gsem2)

          @pl.when(k + 2 < NFULL)
          def _():
            fire_idx(k + 2, sidxN2, didxN2, wbufN2, isemN2)

          compute(sidx, didx, wbuf, rows)
          fire_scatter(rows, didx, ssem)

      wait_scatter(rows1, didx2, ssem1)

      if REM:
        eo = eb + NFULL * CH
        pltpu.sync_copy(src_h.at[pl.ds(eo, REM)], sidx0.at[pl.ds(0, REM)])
        pltpu.sync_copy(dst_h.at[pl.ds(eo, REM)], didx0.at[pl.ds(0, REM)])
        pltpu.sync_copy(w_h.at[pl.ds(eo, REM)], wbuf0.at[pl.ds(0, REM)])
        for i in range(REM // LANES, CH // LANES):
          wbuf0[pl.ds(i * LANES, LANES)] = zero16
        fire_gather(sidx0, rows0, gsem0)
        wait_gather(sidx0, rows0, gsem0)
        compute(sidx0, didx0, wbuf0, rows0)
        pltpu.sync_copy(rows0, sxp.at[didx0], add=True)

    pl.run_scoped(phase_c,
                  pltpu.VMEM((CH,), jnp.int32),
                  pltpu.VMEM((CH,), jnp.int32),
                  pltpu.VMEM((CH,), jnp.float32),
                  pltpu.VMEM((CH,), jnp.int32),
                  pltpu.VMEM((CH,), jnp.int32),
                  pltpu.VMEM((CH,), jnp.float32),
                  pltpu.VMEM((CH,), jnp.int32),
                  pltpu.VMEM((CH,), jnp.int32),
                  pltpu.VMEM((CH,), jnp.float32),
                  pltpu.VMEM((CH,), jnp.float32),
                  pltpu.VMEM((CH, F), jnp.float32),
                  pltpu.VMEM((CH, F), jnp.float32))

    plsc.subcore_barrier()

    rr = s * XPT
    pltpu.sync_copy(sxp.at[pl.ds(rr, XPT)], xpp_h.at[c, pl.ds(rr, XPT)])

  return body(x, src, dst, w)


def _tc_head(xpp, x, dinv_n, Wz, bz, Wlz, blz, Wh, bh, Wlh, blh, Wout, bout):
  N, F = x.shape
  B = 1000
  NB = N // B

  def body(xpp_ref, x_ref, dinv_ref, wz_ref, bz_ref, wlz_ref, blz_ref,
           wh_ref, bh_ref, wlh_ref, blh_ref, wout_ref, bout_ref, y_ref,
           mz_s, cz_s, mh_s, ch_s):
    @pl.when(pl.program_id(0) == 0)
    def _():
      wlz_t = wlz_ref[0:F, :]
      wlh_t = wlh_ref[0:F, :]
      mz_s[...] = jnp.dot(wz_ref[...], wlz_t, preferred_element_type=jnp.float32)
      cz_s[...] = jnp.dot(bz_ref[...], wlz_t, preferred_element_type=jnp.float32) + blz_ref[...]
      mh_s[...] = jnp.dot(wh_ref[...], wlh_t, preferred_element_type=jnp.float32)
      ch_s[...] = jnp.dot(bh_ref[...], wlh_t, preferred_element_type=jnp.float32) + blh_ref[...]

    d = dinv_ref[...]
    xp = xpp_ref[0] + xpp_ref[1] + d * d * x_ref[...]
    z = jax.nn.sigmoid(jnp.dot(xp, mz_s[...], preferred_element_type=jnp.float32) + cz_s[...])
    ht = jnp.tanh(jnp.dot(xp, mh_s[...], preferred_element_type=jnp.float32) + ch_s[...])
    h = (1.0 - z) * ht
    y_ref[...] = jnp.dot(jnp.maximum(h, 0.0), wout_ref[...],
                         preferred_element_type=jnp.float32) + bout_ref[...]

  full = lambda shape: pl.BlockSpec(shape, lambda i: (0,) * len(shape))
  return pl.pallas_call(
      body,
      grid=(NB,),
      in_specs=[
          pl.BlockSpec((NC, B, F), lambda i: (0, i, 0)),
          pl.BlockSpec((B, F), lambda i: (i, 0)),
          pl.BlockSpec((B, 1), lambda i: (i, 0)),
          full((F, F)), full((1, F)), full((2 * F, F)), full((1, F)),
          full((F, F)), full((1, F)), full((2 * F, F)), full((1, F)),
          full((F, 1)), full((1, 1)),
      ],
      out_specs=pl.BlockSpec((B, 1), lambda i: (i, 0)),
      out_shape=jax.ShapeDtypeStruct((N, 1), jnp.float32),
      scratch_shapes=[
          pltpu.VMEM((F, F), jnp.float32),
          pltpu.VMEM((1, F), jnp.float32),
          pltpu.VMEM((F, F), jnp.float32),
          pltpu.VMEM((1, F), jnp.float32),
      ],
  )(xpp, x, dinv_n, Wz, bz, Wlz, blz, Wh, bh, Wlh, blh, Wout, bout)


def kernel(x, edge_index, edge_weight, Wz, bz, Wlz, blz, Wr, br, Wlr, blr,
           Wh, bh, Wlh, blh, Wout, bout):
  N, F = x.shape
  src = edge_index[0]
  dst = edge_index[1]
  xpp, dinv_pad = _sc_propagate(x, src, dst, edge_weight)
  xpp = xpp[:, :N]
  dinv_n = dinv_pad[:N].reshape(N, 1)
  return _tc_head(xpp, x, dinv_n,
                  Wz, bz.reshape(1, F), Wlz, blz.reshape(1, F),
                  Wh, bh.reshape(1, F), Wlh, blh.reshape(1, F),
                  Wout, bout.reshape(1, 1))


# speedup vs baseline: 2.1739x; 1.0981x over previous
import functools

import jax
import jax.numpy as jnp
from jax import lax
from jax.experimental import pallas as pl
from jax.experimental.pallas import tpu as pltpu
from jax.experimental.pallas import tpu_sc as plsc

NC = 2
NS = 16
LANES = 16


def _sc_propagate(x, src, dst, w):
  N, F = x.shape
  E = src.shape[0]
  NPAD = ((N + NS * LANES - 1) // (NS * LANES)) * NS * LANES
  NPT = NPAD // NS
  XPT = ((N + NS * 8 - 1) // (NS * 8)) * 8
  NXP = XPT * NS
  EC = E // NC
  ET = EC // NS
  CH = 128
  NFULL = ET // CH
  REM = ET - NFULL * CH
  assert NFULL % 6 == 0 and REM % 8 == 0
  DEG_PT = E // NS
  DCH = 2000
  assert DCH % LANES == 0 and DEG_PT % DCH == 0
  NDCH = DEG_PT // DCH
  NPR = NPAD // F
  RPT = NPR // NS

  mesh = plsc.VectorSubcoreMesh(core_axis_name="c", subcore_axis_name="s")

  @functools.partial(
      pl.kernel,
      out_type=(
          jax.ShapeDtypeStruct((NC, NXP, F), jnp.float32),
          jax.ShapeDtypeStruct((NPAD,), jnp.float32),
      ),
      mesh=mesh,
      compiler_params=pltpu.CompilerParams(needs_layout_passes=False),
      scratch_types=[
          pltpu.VMEM_SHARED((NPR, F), jnp.float32),
          pltpu.VMEM_SHARED((NPAD,), jnp.float32),
          pltpu.VMEM_SHARED((NXP, F), jnp.float32),
          pltpu.VMEM((RPT, F), jnp.float32),
          pltpu.VMEM((NPT,), jnp.float32),
          pltpu.VMEM((NPAD,), jnp.float32),
          pltpu.VMEM((NPR,), jnp.int32),
          pltpu.SemaphoreType.DMA,
          pltpu.SemaphoreType.DMA,
          pltpu.SemaphoreType.DMA,
          pltpu.SemaphoreType.DMA,
          pltpu.SemaphoreType.DMA,
          pltpu.SemaphoreType.DMA,
          pltpu.SemaphoreType.DMA,
      ],
  )
  def body(x_h, src_h, dst_h, w_h, xpp_h, dinv_h,
           sdeg, sdinv, sxp, dbuf, dacc, dlocal, riota,
           gsem0, ssem0, gsem1, ssem1, isem0, isem1, isem2):
    c = lax.axis_index("c")
    s = lax.axis_index("s")
    zero16 = jnp.zeros((LANES,), jnp.float32)

    @pl.loop(0, NPR // LANES)
    def _(i):
      riota[pl.ds(i * LANES, LANES)] = \
          lax.iota(jnp.int32, LANES) + i * LANES

    assert NDCH % 2 == 0

    def phase_a(ldeg, diA, dwA, diB, dwB):
      dsets = ((diA, dwA, isem0), (diB, dwB, isem1))

      def fire_deg(k, di, dw, dsem):
        base = s * DEG_PT + k * DCH
        pltpu.async_copy(dst_h.at[pl.ds(base, DCH)], di, dsem)
        pltpu.async_copy(w_h.at[pl.ds(base, DCH)], dw, dsem)

      def wait_deg(k, di, dw, dsem):
        base = s * DEG_PT + k * DCH
        pltpu.make_async_copy(dst_h.at[pl.ds(base, DCH)], di, dsem).wait()
        pltpu.make_async_copy(w_h.at[pl.ds(base, DCH)], dw, dsem).wait()

      fire_deg(0, *dsets[0])

      @pl.loop(0, NPR)
      def _(r):
        for j in range(F // LANES):
          ldeg[r, pl.ds(j * LANES, LANES)] = zero16

      pltpu.sync_copy(ldeg.at[pl.ds(0, RPT)], sdeg.at[pl.ds(s * RPT, RPT)])

      @pl.loop(0, NDCH // 2)
      def _(p):
        for u in range(2):
          k = p * 2 + u
          di, dw, dsem = dsets[u]
          wait_deg(k, di, dw, dsem)

          @pl.when(k + 1 < NDCH)
          def _():
            fire_deg(k + 1, *dsets[1 - u])

          @pl.loop(0, DCH // LANES)
          def _(i):
            dv = di[pl.ds(i * LANES, LANES)]
            wv = dw[pl.ds(i * LANES, LANES)]
            plsc.addupdate_scatter(ldeg, [dv >> 7, dv & 127], wv)

      plsc.subcore_barrier()
      pltpu.sync_copy(ldeg, sdeg.at[riota], add=True)

    pl.run_scoped(phase_a,
                  pltpu.VMEM((NPR, F), jnp.float32),
                  pltpu.VMEM((DCH,), jnp.int32),
                  pltpu.VMEM((DCH,), jnp.float32),
                  pltpu.VMEM((DCH,), jnp.int32),
                  pltpu.VMEM((DCH,), jnp.float32))
    plsc.subcore_barrier()

    pltpu.sync_copy(sdeg.at[pl.ds(s * RPT, RPT)], dbuf)

    @pl.loop(0, RPT)
    def _(r):
      for j in range(F // LANES):
        v = dbuf[r, pl.ds(j * LANES, LANES)] + 1.0
        bi = plsc.bitcast(v, jnp.int32)
        y = plsc.bitcast(jnp.int32(0x5F3759DF) - (bi >> 1), jnp.float32)
        y = y * (1.5 - 0.5 * v * y * y)
        y = y * (1.5 - 0.5 * v * y * y)
        y = y * (1.5 - 0.5 * v * y * y)
        dacc[pl.ds(r * F + j * LANES, LANES)] = y

    pltpu.sync_copy(dacc, sdinv.at[pl.ds(s * NPT, NPT)])

    @pl.when(c == 0)
    def _():
      pltpu.sync_copy(dacc, dinv_h.at[pl.ds(s * NPT, NPT)])

    plsc.subcore_barrier()
    pltpu.sync_copy(sdinv, dlocal)

    eb = c * EC + s * ET

    def phase_c(sidx0, didx0, wbuf0, sidx1, didx1, wbuf1,
                sidx2, didx2, wbuf2, nbuf, rows0, rows1):
      idxsets = ((sidx0, didx0, wbuf0, isem0),
                 (sidx1, didx1, wbuf1, isem1),
                 (sidx2, didx2, wbuf2, isem2))
      rowsets = ((rows0, gsem0, ssem0), (rows1, gsem1, ssem1))

      @pl.loop(0, CH)
      def _(r):
        for j in range(F // LANES):
          rows0[r, pl.ds(j * LANES, LANES)] = zero16

      r0 = s * XPT
      for kk in range(XPT // CH):
        pltpu.sync_copy(rows0, sxp.at[pl.ds(r0 + kk * CH, CH)])
      if XPT % CH:
        pltpu.sync_copy(rows0.at[pl.ds(0, XPT % CH)],
                        sxp.at[pl.ds(r0 + (XPT // CH) * CH, XPT % CH)])
      plsc.subcore_barrier()

      def fire_idx(k, sidx, didx, wbuf, isem):
        eo = eb + k * CH
        pltpu.async_copy(src_h.at[pl.ds(eo, CH)], sidx, isem)
        pltpu.async_copy(dst_h.at[pl.ds(eo, CH)], didx, isem)
        pltpu.async_copy(w_h.at[pl.ds(eo, CH)], wbuf, isem)

      def wait_idx(k, sidx, didx, wbuf, isem):
        eo = eb + k * CH
        pltpu.make_async_copy(src_h.at[pl.ds(eo, CH)], sidx, isem).wait()
        pltpu.make_async_copy(dst_h.at[pl.ds(eo, CH)], didx, isem).wait()
        pltpu.make_async_copy(w_h.at[pl.ds(eo, CH)], wbuf, isem).wait()

      def fire_gather(sidx, rows, gsem):
        pltpu.async_copy(x_h.at[sidx], rows, gsem)

      def wait_gather(sidx, rows, gsem):
        pltpu.make_async_copy(x_h.at[sidx], rows, gsem).wait()

      def fire_scatter(rows, didx, ssem):
        pltpu.async_copy(rows, sxp.at[didx], ssem, add=True)

      def wait_scatter(rows, didx, ssem):
        pltpu.make_async_copy(rows, sxp.at[didx], ssem).wait()

      def compute(sidx, didx, wbuf, rows):
        @pl.loop(0, CH // LANES)
        def _(i):
          sl = pl.ds(i * LANES, LANES)
          nbuf[sl] = plsc.load_gather(dlocal, [sidx[sl]]) * wbuf[sl] * \
              plsc.load_gather(dlocal, [didx[sl]])

        @pl.loop(0, CH // LANES)
        def _(g):
          nv = nbuf[pl.ds(g * LANES, LANES)]
          for l in range(LANES):
            nval = nv[l]
            r = g * LANES + l
            for j in range(F // LANES):
              sl = pl.ds(j * LANES, LANES)
              rows[r, sl] = rows[r, sl] * nval

      fire_idx(0, *idxsets[0])
      wait_idx(0, *idxsets[0])
      fire_gather(sidx0, rows0, gsem0)
      fire_idx(1, *idxsets[1])

      @pl.loop(0, NFULL // 6)
      def _(p):
        for u in range(6):
          k = p * 6 + u
          sidx, didx, wbuf, isem = idxsets[u % 3]
          sidxN, didxN, wbufN, isemN = idxsets[(u + 1) % 3]
          sidxN2, didxN2, wbufN2, isemN2 = idxsets[(u + 2) % 3]
          rows, gsem, ssem = rowsets[u % 2]
          rows2, gsem2, ssem2 = rowsets[(u + 1) % 2]

          wait_gather(sidx, rows, gsem)

          @pl.when(k > 0)
          def _():
            wait_scatter(rows2, didxN2, ssem2)

          @pl.when(k + 1 < NFULL)
          def _():
            wait_idx(k + 1, sidxN, didxN, wbufN, isemN)
            fire_gather(sidxN, rows2, gsem2)

          @pl.when(k + 2 < NFULL)
          def _():
            fire_idx(k + 2, sidxN2, didxN2, wbufN2, isemN2)

          compute(sidx, didx, wbuf, rows)
          fire_scatter(rows, didx, ssem)

      wait_scatter(rows1, didx2, ssem1)

      if REM:
        eo = eb + NFULL * CH
        pltpu.sync_copy(src_h.at[pl.ds(eo, REM)], sidx0.at[pl.ds(0, REM)])
        pltpu.sync_copy(dst_h.at[pl.ds(eo, REM)], didx0.at[pl.ds(0, REM)])
        pltpu.sync_copy(w_h.at[pl.ds(eo, REM)], wbuf0.at[pl.ds(0, REM)])
        for i in range(REM // LANES, CH // LANES):
          wbuf0[pl.ds(i * LANES, LANES)] = zero16
        fire_gather(sidx0, rows0, gsem0)
        wait_gather(sidx0, rows0, gsem0)
        compute(sidx0, didx0, wbuf0, rows0)
        pltpu.sync_copy(rows0, sxp.at[didx0], add=True)

    pl.run_scoped(phase_c,
                  pltpu.VMEM((CH,), jnp.int32),
                  pltpu.VMEM((CH,), jnp.int32),
                  pltpu.VMEM((CH,), jnp.float32),
                  pltpu.VMEM((CH,), jnp.int32),
                  pltpu.VMEM((CH,), jnp.int32),
                  pltpu.VMEM((CH,), jnp.float32),
                  pltpu.VMEM((CH,), jnp.int32),
                  pltpu.VMEM((CH,), jnp.int32),
                  pltpu.VMEM((CH,), jnp.float32),
                  pltpu.VMEM((CH,), jnp.float32),
                  pltpu.VMEM((CH, F), jnp.float32),
                  pltpu.VMEM((CH, F), jnp.float32))

    plsc.subcore_barrier()

    rr = s * XPT
    pltpu.sync_copy(sxp.at[pl.ds(rr, XPT)], xpp_h.at[c, pl.ds(rr, XPT)])

  return body(x, src, dst, w)


def _tc_head(xpp, x, dinv_n, Wz, bz, Wlz, blz, Wh, bh, Wlh, blh, Wout, bout):
  N, F = x.shape
  B = 1000
  NB = N // B

  def body(xpp_ref, x_ref, dinv_ref, wz_ref, bz_ref, wlz_ref, blz_ref,
           wh_ref, bh_ref, wlh_ref, blh_ref, wout_ref, bout_ref, y_ref,
           mz_s, cz_s, mh_s, ch_s):
    @pl.when(pl.program_id(0) == 0)
    def _():
      wlz_t = wlz_ref[0:F, :]
      wlh_t = wlh_ref[0:F, :]
      mz_s[...] = jnp.dot(wz_ref[...], wlz_t, preferred_element_type=jnp.float32)
      cz_s[...] = jnp.dot(bz_ref[...], wlz_t, preferred_element_type=jnp.float32) + blz_ref[...]
      mh_s[...] = jnp.dot(wh_ref[...], wlh_t, preferred_element_type=jnp.float32)
      ch_s[...] = jnp.dot(bh_ref[...], wlh_t, preferred_element_type=jnp.float32) + blh_ref[...]

    d = dinv_ref[...]
    xp = xpp_ref[0] + xpp_ref[1] + d * d * x_ref[...]
    z = jax.nn.sigmoid(jnp.dot(xp, mz_s[...], preferred_element_type=jnp.float32) + cz_s[...])
    ht = jnp.tanh(jnp.dot(xp, mh_s[...], preferred_element_type=jnp.float32) + ch_s[...])
    h = (1.0 - z) * ht
    y_ref[...] = jnp.dot(jnp.maximum(h, 0.0), wout_ref[...],
                         preferred_element_type=jnp.float32) + bout_ref[...]

  full = lambda shape: pl.BlockSpec(shape, lambda i: (0,) * len(shape))
  return pl.pallas_call(
      body,
      grid=(NB,),
      in_specs=[
          pl.BlockSpec((NC, B, F), lambda i: (0, i, 0)),
          pl.BlockSpec((B, F), lambda i: (i, 0)),
          pl.BlockSpec((B, 1), lambda i: (i, 0)),
          full((F, F)), full((1, F)), full((2 * F, F)), full((1, F)),
          full((F, F)), full((1, F)), full((2 * F, F)), full((1, F)),
          full((F, 1)), full((1, 1)),
      ],
      out_specs=pl.BlockSpec((B, 1), lambda i: (i, 0)),
      out_shape=jax.ShapeDtypeStruct((N, 1), jnp.float32),
      scratch_shapes=[
          pltpu.VMEM((F, F), jnp.float32),
          pltpu.VMEM((1, F), jnp.float32),
          pltpu.VMEM((F, F), jnp.float32),
          pltpu.VMEM((1, F), jnp.float32),
      ],
  )(xpp, x, dinv_n, Wz, bz, Wlz, blz, Wh, bh, Wlh, blh, Wout, bout)


def kernel(x, edge_index, edge_weight, Wz, bz, Wlz, blz, Wr, br, Wlr, blr,
           Wh, bh, Wlh, blh, Wout, bout):
  N, F = x.shape
  src = edge_index[0]
  dst = edge_index[1]
  xpp, dinv_pad = _sc_propagate(x, src, dst, edge_weight)
  dinv_n = dinv_pad.reshape(-1, 1)
  return _tc_head(xpp, x, dinv_n,
                  Wz, bz.reshape(1, F), Wlz, blz.reshape(1, F),
                  Wh, bh.reshape(1, F), Wlh, blh.reshape(1, F),
                  Wout, bout.reshape(1, 1))


# speedup vs baseline: 2.2817x; 1.0496x over previous
import functools

import jax
import jax.numpy as jnp
from jax import lax
from jax.experimental import pallas as pl
from jax.experimental.pallas import tpu as pltpu
from jax.experimental.pallas import tpu_sc as plsc

NC = 2
NS = 16
LANES = 16


def _sc_propagate(x, ei_flat, w):
  N, F = x.shape
  E = w.shape[0]
  NPAD = ((N + NS * LANES - 1) // (NS * LANES)) * NS * LANES
  NPT = NPAD // NS
  XPT = ((N + NS * 8 - 1) // (NS * 8)) * 8
  NXP = XPT * NS
  EC = E // NC
  ET = EC // NS
  CH = 128
  NFULL = ET // CH
  REM = ET - NFULL * CH
  assert NFULL % 6 == 0 and REM % 8 == 0
  DEG_PT = E // NS
  DCH = 2000
  assert DCH % LANES == 0 and DEG_PT % DCH == 0
  NDCH = DEG_PT // DCH
  NPR = NPAD // F
  RPT = NPR // NS

  mesh = plsc.VectorSubcoreMesh(core_axis_name="c", subcore_axis_name="s")

  @functools.partial(
      pl.kernel,
      out_type=(
          jax.ShapeDtypeStruct((NC, NXP, F), jnp.float32),
          jax.ShapeDtypeStruct((NPAD,), jnp.float32),
      ),
      mesh=mesh,
      compiler_params=pltpu.CompilerParams(needs_layout_passes=False),
      scratch_types=[
          pltpu.VMEM_SHARED((NPR, F), jnp.float32),
          pltpu.VMEM_SHARED((NPAD,), jnp.float32),
          pltpu.VMEM_SHARED((NXP, F), jnp.float32),
          pltpu.VMEM((RPT, F), jnp.float32),
          pltpu.VMEM((NPT,), jnp.float32),
          pltpu.VMEM((NPAD,), jnp.float32),
          pltpu.VMEM((NPR,), jnp.int32),
          pltpu.SemaphoreType.DMA,
          pltpu.SemaphoreType.DMA,
          pltpu.SemaphoreType.DMA,
          pltpu.SemaphoreType.DMA,
          pltpu.SemaphoreType.DMA,
          pltpu.SemaphoreType.DMA,
          pltpu.SemaphoreType.DMA,
      ],
  )
  def body(x_h, ei_h, w_h, xpp_h, dinv_h,
           sdeg, sdinv, sxp, dbuf, dacc, dlocal, riota,
           gsem0, ssem0, gsem1, ssem1, isem0, isem1, isem2):
    c = lax.axis_index("c")
    s = lax.axis_index("s")
    zero16 = jnp.zeros((LANES,), jnp.float32)

    @pl.loop(0, NPR // LANES)
    def _(i):
      riota[pl.ds(i * LANES, LANES)] = \
          lax.iota(jnp.int32, LANES) + i * LANES

    assert NDCH % 2 == 0

    def phase_a(ldeg, diA, dwA, diB, dwB):
      dsets = ((diA, dwA, isem0), (diB, dwB, isem1))

      def fire_deg(k, di, dw, dsem):
        base = s * DEG_PT + k * DCH
        pltpu.async_copy(ei_h.at[pl.ds(E + base, DCH)], di, dsem)
        pltpu.async_copy(w_h.at[pl.ds(base, DCH)], dw, dsem)

      def wait_deg(k, di, dw, dsem):
        base = s * DEG_PT + k * DCH
        pltpu.make_async_copy(ei_h.at[pl.ds(E + base, DCH)], di, dsem).wait()
        pltpu.make_async_copy(w_h.at[pl.ds(base, DCH)], dw, dsem).wait()

      fire_deg(0, *dsets[0])

      @pl.loop(0, NPR)
      def _(r):
        for j in range(F // LANES):
          ldeg[r, pl.ds(j * LANES, LANES)] = zero16

      pltpu.sync_copy(ldeg.at[pl.ds(0, RPT)], sdeg.at[pl.ds(s * RPT, RPT)])

      @pl.loop(0, NDCH // 2)
      def _(p):
        for u in range(2):
          k = p * 2 + u
          di, dw, dsem = dsets[u]
          wait_deg(k, di, dw, dsem)

          @pl.when(k + 1 < NDCH)
          def _():
            fire_deg(k + 1, *dsets[1 - u])

          @pl.loop(0, DCH // LANES)
          def _(i):
            dv = di[pl.ds(i * LANES, LANES)]
            wv = dw[pl.ds(i * LANES, LANES)]
            plsc.addupdate_scatter(ldeg, [dv >> 7, dv & 127], wv)

      plsc.subcore_barrier()
      pltpu.sync_copy(ldeg, sdeg.at[riota], add=True)

    pl.run_scoped(phase_a,
                  pltpu.VMEM((NPR, F), jnp.float32),
                  pltpu.VMEM((DCH,), jnp.int32),
                  pltpu.VMEM((DCH,), jnp.float32),
                  pltpu.VMEM((DCH,), jnp.int32),
                  pltpu.VMEM((DCH,), jnp.float32))
    plsc.subcore_barrier()

    pltpu.sync_copy(sdeg.at[pl.ds(s * RPT, RPT)], dbuf)

    @pl.loop(0, RPT)
    def _(r):
      for j in range(F // LANES):
        v = dbuf[r, pl.ds(j * LANES, LANES)] + 1.0
        bi = plsc.bitcast(v, jnp.int32)
        y = plsc.bitcast(jnp.int32(0x5F3759DF) - (bi >> 1), jnp.float32)
        y = y * (1.5 - 0.5 * v * y * y)
        y = y * (1.5 - 0.5 * v * y * y)
        y = y * (1.5 - 0.5 * v * y * y)
        dacc[pl.ds(r * F + j * LANES, LANES)] = y

    pltpu.sync_copy(dacc, sdinv.at[pl.ds(s * NPT, NPT)])

    @pl.when(c == 0)
    def _():
      pltpu.sync_copy(dacc, dinv_h.at[pl.ds(s * NPT, NPT)])

    plsc.subcore_barrier()
    pltpu.sync_copy(sdinv, dlocal)

    eb = c * EC + s * ET

    def phase_c(sidx0, didx0, wbuf0, sidx1, didx1, wbuf1,
                sidx2, didx2, wbuf2, nbuf, rows0, rows1):
      idxsets = ((sidx0, didx0, wbuf0, isem0),
                 (sidx1, didx1, wbuf1, isem1),
                 (sidx2, didx2, wbuf2, isem2))
      rowsets = ((rows0, gsem0, ssem0), (rows1, gsem1, ssem1))

      @pl.loop(0, CH)
      def _(r):
        for j in range(F // LANES):
          rows0[r, pl.ds(j * LANES, LANES)] = zero16

      r0 = s * XPT
      for kk in range(XPT // CH):
        pltpu.sync_copy(rows0, sxp.at[pl.ds(r0 + kk * CH, CH)])
      if XPT % CH:
        pltpu.sync_copy(rows0.at[pl.ds(0, XPT % CH)],
                        sxp.at[pl.ds(r0 + (XPT // CH) * CH, XPT % CH)])
      plsc.subcore_barrier()

      def fire_idx(k, sidx, didx, wbuf, isem):
        eo = eb + k * CH
        pltpu.async_copy(ei_h.at[pl.ds(eo, CH)], sidx, isem)
        pltpu.async_copy(ei_h.at[pl.ds(E + eo, CH)], didx, isem)
        pltpu.async_copy(w_h.at[pl.ds(eo, CH)], wbuf, isem)

      def wait_idx(k, sidx, didx, wbuf, isem):
        eo = eb + k * CH
        pltpu.make_async_copy(ei_h.at[pl.ds(eo, CH)], sidx, isem).wait()
        pltpu.make_async_copy(ei_h.at[pl.ds(E + eo, CH)], didx, isem).wait()
        pltpu.make_async_copy(w_h.at[pl.ds(eo, CH)], wbuf, isem).wait()

      def fire_gather(sidx, rows, gsem):
        pltpu.async_copy(x_h.at[sidx], rows, gsem)

      def wait_gather(sidx, rows, gsem):
        pltpu.make_async_copy(x_h.at[sidx], rows, gsem).wait()

      def fire_scatter(rows, didx, ssem):
        pltpu.async_copy(rows, sxp.at[didx], ssem, add=True)

      def wait_scatter(rows, didx, ssem):
        pltpu.make_async_copy(rows, sxp.at[didx], ssem).wait()

      def compute(sidx, didx, wbuf, rows):
        @pl.loop(0, CH // LANES)
        def _(i):
          sl = pl.ds(i * LANES, LANES)
          nbuf[sl] = plsc.load_gather(dlocal, [sidx[sl]]) * wbuf[sl] * \
              plsc.load_gather(dlocal, [didx[sl]])

        @pl.loop(0, CH // LANES)
        def _(g):
          nv = nbuf[pl.ds(g * LANES, LANES)]
          for l in range(LANES):
            nval = nv[l]
            r = g * LANES + l
            for j in range(F // LANES):
              sl = pl.ds(j * LANES, LANES)
              rows[r, sl] = rows[r, sl] * nval

      fire_idx(0, *idxsets[0])
      wait_idx(0, *idxsets[0])
      fire_gather(sidx0, rows0, gsem0)
      fire_idx(1, *idxsets[1])

      @pl.loop(0, NFULL // 6)
      def _(p):
        for u in range(6):
          k = p * 6 + u
          sidx, didx, wbuf, isem = idxsets[u % 3]
          sidxN, didxN, wbufN, isemN = idxsets[(u + 1) % 3]
          sidxN2, didxN2, wbufN2, isemN2 = idxsets[(u + 2) % 3]
          rows, gsem, ssem = rowsets[u % 2]
          rows2, gsem2, ssem2 = rowsets[(u + 1) % 2]

          wait_gather(sidx, rows, gsem)

          @pl.when(k > 0)
          def _():
            wait_scatter(rows2, didxN2, ssem2)

          @pl.when(k + 1 < NFULL)
          def _():
            wait_idx(k + 1, sidxN, didxN, wbufN, isemN)
            fire_gather(sidxN, rows2, gsem2)

          @pl.when(k + 2 < NFULL)
          def _():
            fire_idx(k + 2, sidxN2, didxN2, wbufN2, isemN2)

          compute(sidx, didx, wbuf, rows)
          fire_scatter(rows, didx, ssem)

      wait_scatter(rows1, didx2, ssem1)

      if REM:
        eo = eb + NFULL * CH
        pltpu.sync_copy(ei_h.at[pl.ds(eo, REM)], sidx0.at[pl.ds(0, REM)])
        pltpu.sync_copy(ei_h.at[pl.ds(E + eo, REM)], didx0.at[pl.ds(0, REM)])
        pltpu.sync_copy(w_h.at[pl.ds(eo, REM)], wbuf0.at[pl.ds(0, REM)])
        for i in range(REM // LANES, CH // LANES):
          wbuf0[pl.ds(i * LANES, LANES)] = zero16
        fire_gather(sidx0, rows0, gsem0)
        wait_gather(sidx0, rows0, gsem0)
        compute(sidx0, didx0, wbuf0, rows0)
        pltpu.sync_copy(rows0, sxp.at[didx0], add=True)

    pl.run_scoped(phase_c,
                  pltpu.VMEM((CH,), jnp.int32),
                  pltpu.VMEM((CH,), jnp.int32),
                  pltpu.VMEM((CH,), jnp.float32),
                  pltpu.VMEM((CH,), jnp.int32),
                  pltpu.VMEM((CH,), jnp.int32),
                  pltpu.VMEM((CH,), jnp.float32),
                  pltpu.VMEM((CH,), jnp.int32),
                  pltpu.VMEM((CH,), jnp.int32),
                  pltpu.VMEM((CH,), jnp.float32),
                  pltpu.VMEM((CH,), jnp.float32),
                  pltpu.VMEM((CH, F), jnp.float32),
                  pltpu.VMEM((CH, F), jnp.float32))

    plsc.subcore_barrier()

    rr = s * XPT
    pltpu.sync_copy(sxp.at[pl.ds(rr, XPT)], xpp_h.at[c, pl.ds(rr, XPT)])

  return body(x, ei_flat, w)


def _tc_head(xpp, x, dinv_n, Wz, bz, Wlz, blz, Wh, bh, Wlh, blh, Wout, bout):
  N, F = x.shape
  B = 1000
  NB = N // B

  def body(xpp_ref, x_ref, dinv_ref, wz_ref, bz_ref, wlz_ref, blz_ref,
           wh_ref, bh_ref, wlh_ref, blh_ref, wout_ref, bout_ref, y_ref,
           mz_s, cz_s, mh_s, ch_s):
    @pl.when(pl.program_id(0) == 0)
    def _():
      wlz_t = wlz_ref[0:F, :]
      wlh_t = wlh_ref[0:F, :]
      mz_s[...] = jnp.dot(wz_ref[...], wlz_t, preferred_element_type=jnp.float32)
      cz_s[...] = jnp.dot(bz_ref[...], wlz_t, preferred_element_type=jnp.float32) + blz_ref[...]
      mh_s[...] = jnp.dot(wh_ref[...], wlh_t, preferred_element_type=jnp.float32)
      ch_s[...] = jnp.dot(bh_ref[...], wlh_t, preferred_element_type=jnp.float32) + blh_ref[...]

    d = dinv_ref[...]
    xp = xpp_ref[0] + xpp_ref[1] + d * d * x_ref[...]
    z = jax.nn.sigmoid(jnp.dot(xp, mz_s[...], preferred_element_type=jnp.float32) + cz_s[...])
    ht = jnp.tanh(jnp.dot(xp, mh_s[...], preferred_element_type=jnp.float32) + ch_s[...])
    h = (1.0 - z) * ht
    y_ref[...] = jnp.dot(jnp.maximum(h, 0.0), wout_ref[...],
                         preferred_element_type=jnp.float32) + bout_ref[...]

  full = lambda shape: pl.BlockSpec(shape, lambda i: (0,) * len(shape))
  return pl.pallas_call(
      body,
      grid=(NB,),
      in_specs=[
          pl.BlockSpec((NC, B, F), lambda i: (0, i, 0)),
          pl.BlockSpec((B, F), lambda i: (i, 0)),
          pl.BlockSpec((B, 1), lambda i: (i, 0)),
          full((F, F)), full((1, F)), full((2 * F, F)), full((1, F)),
          full((F, F)), full((1, F)), full((2 * F, F)), full((1, F)),
          full((F, 1)), full((1, 1)),
      ],
      out_specs=pl.BlockSpec((B, 1), lambda i: (i, 0)),
      out_shape=jax.ShapeDtypeStruct((N, 1), jnp.float32),
      scratch_shapes=[
          pltpu.VMEM((F, F), jnp.float32),
          pltpu.VMEM((1, F), jnp.float32),
          pltpu.VMEM((F, F), jnp.float32),
          pltpu.VMEM((1, F), jnp.float32),
      ],
  )(xpp, x, dinv_n, Wz, bz, Wlz, blz, Wh, bh, Wlh, blh, Wout, bout)


def kernel(x, edge_index, edge_weight, Wz, bz, Wlz, blz, Wr, br, Wlr, blr,
           Wh, bh, Wlh, blh, Wout, bout):
  N, F = x.shape
  xpp, dinv_pad = _sc_propagate(x, edge_index.reshape(-1), edge_weight)
  dinv_n = dinv_pad.reshape(-1, 1)
  return _tc_head(xpp, x, dinv_n,
                  Wz, bz.reshape(1, F), Wlz, blz.reshape(1, F),
                  Wh, bh.reshape(1, F), Wlh, blh.reshape(1, F),
                  Wout, bout.reshape(1, 1))
